# baseline (speedup 1.0000x reference)
import functools

import jax
import jax.numpy as jnp
from jax import lax
from jax.experimental import pallas as pl
from jax.experimental.pallas import tpu as pltpu
from jax.experimental.pallas import tpu_sc as plsc

N = 65536
D = 128
NT = 32
NCT = 64
L = 16
NC = 2
NS = 16
NW = NC * NS
NPW = N // NW
C = 128
NCHUNKS = NPW // C

_mesh = plsc.VectorSubcoreMesh(core_axis_name="c", subcore_axis_name="s")


@functools.partial(
    pl.kernel,
    mesh=_mesh,
    out_type=[
        jax.ShapeDtypeStruct((N * D,), jnp.float32),
        jax.ShapeDtypeStruct((N * 3 * D,), jnp.float32),
    ],
    compiler_params=pltpu.CompilerParams(needs_layout_passes=False),
    scratch_types=[
        pltpu.VMEM((NT * D,), jnp.float32),
        pltpu.VMEM((NCT * D,), jnp.float32),
        pltpu.VMEM((NT * 3 * D,), jnp.float32),
        pltpu.VMEM((C + L,), jnp.int32),
        pltpu.VMEM((C + L,), jnp.int32),
        pltpu.VMEM((C * 9 + L,), jnp.float32),
        pltpu.VMEM((C * D,), jnp.float32),
        pltpu.VMEM((C * 3 * D,), jnp.float32),
    ],
)
def _sc_embed(nt_hbm, cid_hbm, rot_hbm, ts_hbm, cs_hbm, tvp_hbm,
              s_hbm, v_hbm,
              ts_v, cs_v, tvp_v, nt_v, cid_v, rot_v, s_buf, v_buf):
    wid = lax.axis_index("s") * NC + lax.axis_index("c")
    base = wid * NPW

    pltpu.sync_copy(ts_hbm, ts_v)
    pltpu.sync_copy(cs_hbm, cs_v)
    pltpu.sync_copy(tvp_hbm, tvp_v)

    lane3 = lax.iota(jnp.int32, L) * 3

    def chunk_body(g, carry):
        nbase = base + g * C
        pltpu.sync_copy(nt_hbm.at[pl.ds(nbase, C)], nt_v.at[pl.ds(0, C)])
        pltpu.sync_copy(cid_hbm.at[pl.ds(nbase, C)], cid_v.at[pl.ds(0, C)])
        pltpu.sync_copy(rot_hbm.at[pl.ds(nbase * 9, C * 9)],
                        rot_v.at[pl.ds(0, C * 9)])

        def node_body(n, carry2):
            nt = nt_v[pl.ds(n, L)][0]
            cid = cid_v[pl.ds(n, L)][0]
            tsb = nt * D
            csb = cid * D
            for cb in range(D // L):
                a = ts_v[pl.ds(tsb + cb * L, L)]
                b = cs_v[pl.ds(csb + cb * L, L)]
                s_buf[pl.ds(n * D + cb * L, L)] = a + b
            rv = rot_v[pl.ds(n * 9, L)]
            r = [[rv[3 * i + j] for j in range(3)] for i in range(3)]
            tvb = nt * (3 * D)
            vb = n * (3 * D)
            for db in range(D // L):
                p = [tvp_v[pl.ds(tvb + j * D + db * L, L)] for j in range(3)]
                for i in range(3):
                    o = r[i][0] * p[0] + r[i][1] * p[1] + r[i][2] * p[2]
                    plsc.store_scatter(v_buf, [lane3 + (vb + 3 * L * db + i)], o)
            return carry2

        lax.fori_loop(0, C, node_body, 0)
        pltpu.sync_copy(s_buf, s_hbm.at[pl.ds(nbase * D, C * D)])
        pltpu.sync_copy(v_buf, v_hbm.at[pl.ds(nbase * 3 * D, C * 3 * D)])
        return carry

    lax.fori_loop(0, NCHUNKS, chunk_body, 0)


def kernel(node_type, rotmat, chain_id, type2scalar, type2vec, chain2scalar):
    nt = node_type.astype(jnp.int32)
    cid = chain_id.astype(jnp.int32)
    rot = rotmat.reshape(N * 9)
    ts = type2scalar.reshape(NT * D)
    cs = chain2scalar.reshape(NCT * D)
    tvp = type2vec.reshape(NT, D, 3).transpose(0, 2, 1).reshape(NT * 3 * D)
    s_flat, v_flat = _sc_embed(nt, cid, rot, ts, cs, tvp)
    return s_flat.reshape(N, D), v_flat.reshape(N, D, 3)


# speedup vs baseline: 1.0291x; 1.0291x over previous
import functools

import jax
import jax.numpy as jnp
from jax import lax
from jax.experimental import pallas as pl
from jax.experimental.pallas import tpu as pltpu
from jax.experimental.pallas import tpu_sc as plsc

N = 65536
D = 128
NT = 32
NCT = 64
L = 16
NC = 2
NS = 16
NW = NC * NS
NPW = N // NW
C = 128
NCHUNKS = NPW // C

_mesh = plsc.VectorSubcoreMesh(core_axis_name="c", subcore_axis_name="s")


@functools.partial(
    pl.kernel,
    mesh=_mesh,
    out_type=[
        jax.ShapeDtypeStruct((N * D,), jnp.float32),
        jax.ShapeDtypeStruct((N * 3 * D,), jnp.float32),
    ],
    compiler_params=pltpu.CompilerParams(needs_layout_passes=False),
    scratch_types=[
        pltpu.VMEM((NT * D,), jnp.float32),
        pltpu.VMEM((NCT * D,), jnp.float32),
        pltpu.VMEM((NT * 3 * D,), jnp.float32),
        pltpu.VMEM((C + L,), jnp.int32),
        pltpu.VMEM((C + L,), jnp.int32),
        pltpu.VMEM((C * 9 + L,), jnp.float32),
        pltpu.VMEM((C * D,), jnp.float32),
        pltpu.VMEM((C * 3 * D,), jnp.float32),
    ],
)
def _sc_embed(nt_hbm, cid_hbm, rot_hbm, ts_hbm, cs_hbm, tvp_hbm,
              s_hbm, v_hbm,
              ts_v, cs_v, tvp_v, nt_v, cid_v, rot_v, s_buf, v_buf):
    wid = lax.axis_index("s") * NC + lax.axis_index("c")
    base = wid * NPW

    pltpu.sync_copy(ts_hbm, ts_v)
    pltpu.sync_copy(cs_hbm, cs_v)
    pltpu.sync_copy(tvp_hbm, tvp_v)

    lane3 = lax.iota(jnp.int32, L) * 3

    def chunk_body(g, carry):
        nbase = base + g * C
        pltpu.sync_copy(nt_hbm.at[pl.ds(nbase, C)], nt_v.at[pl.ds(0, C)])
        pltpu.sync_copy(cid_hbm.at[pl.ds(nbase, C)], cid_v.at[pl.ds(0, C)])
        pltpu.sync_copy(rot_hbm.at[pl.ds(nbase * 9, C * 9)],
                        rot_v.at[pl.ds(0, C * 9)])

        @plsc.parallel_loop(0, C, 1, unroll=4)
        def node_body(n):
            nt = nt_v[pl.ds(n, L)][0]
            cid = cid_v[pl.ds(n, L)][0]
            tsb = nt * D
            csb = cid * D
            for cb in range(D // L):
                a = ts_v[pl.ds(tsb + cb * L, L)]
                b = cs_v[pl.ds(csb + cb * L, L)]
                s_buf[pl.ds(n * D + cb * L, L)] = a + b
            rv = rot_v[pl.ds(n * 9, L)]
            r = [[rv[3 * i + j] for j in range(3)] for i in range(3)]
            tvb = nt * (3 * D)
            vb = n * (3 * D)
            for db in range(D // L):
                p = [tvp_v[pl.ds(tvb + j * D + db * L, L)] for j in range(3)]
                for i in range(3):
                    o = r[i][0] * p[0] + r[i][1] * p[1] + r[i][2] * p[2]
                    plsc.store_scatter(v_buf, [lane3 + (vb + 3 * L * db + i)], o)

        pltpu.sync_copy(s_buf, s_hbm.at[pl.ds(nbase * D, C * D)])
        pltpu.sync_copy(v_buf, v_hbm.at[pl.ds(nbase * 3 * D, C * 3 * D)])
        return carry

    lax.fori_loop(0, NCHUNKS, chunk_body, 0)


def kernel(node_type, rotmat, chain_id, type2scalar, type2vec, chain2scalar):
    nt = node_type.astype(jnp.int32)
    cid = chain_id.astype(jnp.int32)
    rot = rotmat.reshape(N * 9)
    ts = type2scalar.reshape(NT * D)
    cs = chain2scalar.reshape(NCT * D)
    tvp = type2vec.reshape(NT, D, 3).transpose(0, 2, 1).reshape(NT * 3 * D)
    s_flat, v_flat = _sc_embed(nt, cid, rot, ts, cs, tvp)
    return s_flat.reshape(N, D), v_flat.reshape(N, D, 3)


# speedup vs baseline: 7.9295x; 7.7055x over previous
import functools

import jax
import jax.numpy as jnp
from jax import lax
from jax.experimental import pallas as pl
from jax.experimental.pallas import tpu as pltpu
from jax.experimental.pallas import tpu_sc as plsc

N = 65536
D = 128
NT = 32
NCT = 64
L = 16
NC = 2
NS = 16
NW = NC * NS
NPW = N // NW
C = 128
NCHUNKS = NPW // C

_mesh = plsc.VectorSubcoreMesh(core_axis_name="c", subcore_axis_name="s")


@functools.partial(
    pl.kernel,
    mesh=_mesh,
    out_type=[
        jax.ShapeDtypeStruct((N * D,), jnp.float32),
        jax.ShapeDtypeStruct((N * 3 * D,), jnp.float32),
    ],
    compiler_params=pltpu.CompilerParams(needs_layout_passes=False),
    scratch_types=[
        pltpu.VMEM((NT * D,), jnp.float32),
        pltpu.VMEM((NCT * D,), jnp.float32),
        pltpu.VMEM((NT * 3 * D,), jnp.float32),
        pltpu.VMEM((C + L,), jnp.int32),
        pltpu.VMEM((C + L,), jnp.int32),
        pltpu.VMEM((C * 9 + L,), jnp.float32),
        pltpu.VMEM((C * D,), jnp.float32),
        pltpu.VMEM((C * 3 * D,), jnp.float32),
    ],
)
def _sc_embed(nt_hbm, cid_hbm, rot_hbm, ts_hbm, cs_hbm, tvp_hbm,
              s_hbm, v_hbm,
              ts_v, cs_v, tvp_v, nt_v, cid_v, rot_v, s_buf, v_buf):
    wid = lax.axis_index("s") * NC + lax.axis_index("c")
    base = wid * NPW

    pltpu.sync_copy(ts_hbm, ts_v)
    pltpu.sync_copy(cs_hbm, cs_v)
    pltpu.sync_copy(tvp_hbm, tvp_v)

    def chunk_body(g, carry):
        nbase = base + g * C
        pltpu.sync_copy(nt_hbm.at[pl.ds(nbase, C)], nt_v.at[pl.ds(0, C)])
        pltpu.sync_copy(cid_hbm.at[pl.ds(nbase, C)], cid_v.at[pl.ds(0, C)])
        pltpu.sync_copy(rot_hbm.at[pl.ds(nbase * 9, C * 9)],
                        rot_v.at[pl.ds(0, C * 9)])

        @plsc.parallel_loop(0, C, 1, unroll=4)
        def node_body(n):
            nt = nt_v[pl.ds(n, L)][0]
            cid = cid_v[pl.ds(n, L)][0]
            tsb = nt * D
            csb = cid * D
            for cb in range(D // L):
                a = ts_v[pl.ds(tsb + cb * L, L)]
                b = cs_v[pl.ds(csb + cb * L, L)]
                s_buf[pl.ds(n * D + cb * L, L)] = a + b
            rv = rot_v[pl.ds(n * 9, L)]
            r = [[rv[3 * i + j] for j in range(3)] for i in range(3)]
            tvb = nt * (3 * D)
            vb = n * (3 * D)
            for db in range(D // L):
                p = [tvp_v[pl.ds(tvb + j * D + db * L, L)] for j in range(3)]
                for i in range(3):
                    o = r[i][0] * p[0] + r[i][1] * p[1] + r[i][2] * p[2]
                    v_buf[pl.ds(vb + i * D + db * L, L)] = o

        pltpu.sync_copy(s_buf, s_hbm.at[pl.ds(nbase * D, C * D)])
        pltpu.sync_copy(v_buf, v_hbm.at[pl.ds(nbase * 3 * D, C * 3 * D)])
        return carry

    lax.fori_loop(0, NCHUNKS, chunk_body, 0)


def kernel(node_type, rotmat, chain_id, type2scalar, type2vec, chain2scalar):
    nt = node_type.astype(jnp.int32)
    cid = chain_id.astype(jnp.int32)
    rot = rotmat.reshape(N * 9)
    ts = type2scalar.reshape(NT * D)
    cs = chain2scalar.reshape(NCT * D)
    tvp = type2vec.reshape(NT, D, 3).transpose(0, 2, 1).reshape(NT * 3 * D)
    s_flat, v_flat = _sc_embed(nt, cid, rot, ts, cs, tvp)
    return s_flat.reshape(N, D), v_flat.reshape(N, 3, D).swapaxes(1, 2)


# speedup vs baseline: 9.1452x; 1.1533x over previous
import functools

import jax
import jax.numpy as jnp
from jax import lax
from jax.experimental import pallas as pl
from jax.experimental.pallas import tpu as pltpu
from jax.experimental.pallas import tpu_sc as plsc

N = 65536
D = 128
NT = 32
NCT = 64
L = 16
NC = 2
NS = 16
NW = NC * NS
NPW = N // NW
C = 128
NCHUNKS = NPW // C

_mesh = plsc.VectorSubcoreMesh(core_axis_name="c", subcore_axis_name="s")


@functools.partial(
    pl.kernel,
    mesh=_mesh,
    out_type=[
        jax.ShapeDtypeStruct((N * D,), jnp.float32),
        jax.ShapeDtypeStruct((3 * N * D,), jnp.float32),
    ],
    compiler_params=pltpu.CompilerParams(needs_layout_passes=False),
    scratch_types=[
        pltpu.VMEM((NT * D,), jnp.float32),
        pltpu.VMEM((NCT * D,), jnp.float32),
        pltpu.VMEM((NT * 3 * D,), jnp.float32),
        pltpu.VMEM((C,), jnp.int32),
        pltpu.VMEM((C,), jnp.int32),
        pltpu.VMEM((9 * C,), jnp.float32),
        pltpu.VMEM((C * D,), jnp.float32),
        pltpu.VMEM((3 * C * D,), jnp.float32),
    ],
)
def _sc_embed(nt_hbm, cid_hbm, rot_hbm, ts_hbm, cs_hbm, tvp_hbm,
              s_hbm, v_hbm,
              ts_v, cs_v, tvp_v, nt_v, cid_v, rot_v, s_buf, v_buf):
    wid = lax.axis_index("s") * NC + lax.axis_index("c")
    base = wid * NPW

    pltpu.sync_copy(ts_hbm, ts_v)
    pltpu.sync_copy(cs_hbm, cs_v)
    pltpu.sync_copy(tvp_hbm, tvp_v)

    def chunk_body(g, carry):
        nbase = base + g * C
        pltpu.sync_copy(nt_hbm.at[pl.ds(nbase, C)], nt_v)
        pltpu.sync_copy(cid_hbm.at[pl.ds(nbase, C)], cid_v)
        for k in range(9):
            pltpu.sync_copy(rot_hbm.at[pl.ds(k * N + nbase, C)],
                            rot_v.at[pl.ds(k * C, C)])

        @plsc.parallel_loop(0, C // L, 1)
        def group_body(nb):
            gb = nb * L
            nt16 = nt_v[pl.ds(gb, L)]
            cid16 = cid_v[pl.ds(gb, L)]
            rv = [rot_v[pl.ds(k * C + gb, L)] for k in range(9)]
            for m in range(L):
                n = gb + m
                tsb = nt16[m] * D
                csb = cid16[m] * D
                for cb in range(D // L):
                    a = ts_v[pl.ds(tsb + cb * L, L)]
                    b = cs_v[pl.ds(csb + cb * L, L)]
                    s_buf[pl.ds(n * D + cb * L, L)] = a + b
                r = [rv[k][m] for k in range(9)]
                tvb = nt16[m] * (3 * D)
                for db in range(D // L):
                    p = [tvp_v[pl.ds(tvb + j * D + db * L, L)]
                         for j in range(3)]
                    for i in range(3):
                        o = r[3 * i] * p[0] + r[3 * i + 1] * p[1] \
                            + r[3 * i + 2] * p[2]
                        v_buf[pl.ds(i * (C * D) + n * D + db * L, L)] = o

        pltpu.sync_copy(s_buf, s_hbm.at[pl.ds(nbase * D, C * D)])
        for i in range(3):
            pltpu.sync_copy(v_buf.at[pl.ds(i * (C * D), C * D)],
                            v_hbm.at[pl.ds(i * (N * D) + nbase * D, C * D)])
        return carry

    lax.fori_loop(0, NCHUNKS, chunk_body, 0)


def kernel(node_type, rotmat, chain_id, type2scalar, type2vec, chain2scalar):
    nt = node_type.astype(jnp.int32)
    cid = chain_id.astype(jnp.int32)
    rot = rotmat.transpose(1, 2, 0).reshape(9 * N)
    ts = type2scalar.reshape(NT * D)
    cs = chain2scalar.reshape(NCT * D)
    tvp = type2vec.reshape(NT, D, 3).transpose(0, 2, 1).reshape(NT * 3 * D)
    s_flat, v_flat = _sc_embed(nt, cid, rot, ts, cs, tvp)
    return (s_flat.reshape(N, D),
            v_flat.reshape(3, N, D).transpose(1, 2, 0))
